# baseline (speedup 1.0000x reference)
import functools

import jax
import jax.numpy as jnp
from jax.experimental import pallas as pl

F32 = jnp.float32
BF16 = jnp.bfloat16


def _dot(a, b):
    return jnp.dot(a.astype(BF16), b.astype(BF16), preferred_element_type=F32)


def _ln(x, g, b):
    m = x.mean(-1, keepdims=True)
    v = ((x - m) ** 2).mean(-1, keepdims=True)
    return (x - m) / jnp.sqrt(v + 1e-6) * g + b


def _fwd_kernel(
    ids_m_ref, ids_h_ref, pos_m_ref, pos_h_ref, emb_ref,
    wq_ref, bq_ref, wk_ref, bk_ref, wv_ref, bv_ref, wo_ref, bo_ref,
    g1_ref, b1_ref, g2_ref, b2_ref,
    w1_ref, bf1_ref, w2_ref, bf2_ref,
    lmw_ref, lmb_ref, l1w_ref, l1b_ref, chw_ref, chb_ref,
    lbl_oh_ref, w_ref,
    out_ref,
    *, T, CH, H, DH, V, NL,
):
    s_blk = pl.program_id(0)
    TE = T + CH

    ids_ext = jnp.concatenate([ids_h_ref[0], ids_m_ref[0]], axis=0)
    pos_ext = jnp.concatenate([pos_h_ref[0], pos_m_ref[0]], axis=0)
    onehot = (ids_ext == jax.lax.broadcasted_iota(jnp.int32, (TE, V), 1)
              ).astype(BF16)
    x_ext = _dot(onehot, emb_ref[...]) + pos_ext

    h1 = _ln(x_ext, g1_ref[...], b1_ref[...])
    q_all = _dot(h1[CH:], wq_ref[...]) + bq_ref[...]
    k_all = _dot(h1, wk_ref[...]) + bk_ref[...]
    v_all = _dot(h1, wv_ref[...]) + bv_ref[...]

    ti = jax.lax.broadcasted_iota(jnp.int32, (T, TE), 0)
    ji = jax.lax.broadcasted_iota(jnp.int32, (T, TE), 1)
    tc = ti // CH
    jc = ji // CH
    band = (jc == tc) | (jc == tc + 1)
    causal = ji <= ti + CH
    first = (s_blk * (T // CH) + tc) > 0
    mask = band & causal & (first | (ji >= CH))

    scale = 1.0 / (DH ** 0.5)
    outs = []
    for h in range(H):
        sl = slice(h * DH, (h + 1) * DH)
        qh = q_all[:, sl]
        kh = k_all[:, sl]
        vh = v_all[:, sl]
        sc = jax.lax.dot_general(qh.astype(BF16), kh.astype(BF16),
                                 (((1,), (1,)), ((), ())),
                                 preferred_element_type=F32) * scale
        sc = jnp.where(mask, sc, -1e9)
        mx = jnp.max(sc, axis=-1, keepdims=True)
        p = jnp.exp(sc - mx)
        p = p / jnp.sum(p, axis=-1, keepdims=True)
        outs.append(_dot(p, vh))
    o = jnp.concatenate(outs, axis=-1)

    x = x_ext[CH:] + _dot(o, wo_ref[...]) + bo_ref[...]

    h2 = _ln(x, g2_ref[...], b2_ref[...])
    ff = jnp.maximum(_dot(h2, w1_ref[...]) + bf1_ref[...], 0.0)
    x = x + _dot(ff, w2_ref[...]) + bf2_ref[...]

    lm = _dot(x, lmw_ref[...]) + lmb_ref[...]
    o1 = jnp.maximum(_dot(lm, l1w_ref[...]) + l1b_ref[...], 0.0)
    logits = _dot(o1, chw_ref[...]) + chb_ref[...]

    mx = jnp.max(logits, axis=-1, keepdims=True)
    lse = mx + jnp.log(jnp.sum(jnp.exp(logits - mx), axis=-1, keepdims=True))
    sel = jnp.sum(lbl_oh_ref[0] * logits, axis=-1, keepdims=True)
    wcol = w_ref[0]
    nll_sum = jnp.sum((lse - sel) * wcol)
    w_sum = jnp.sum(wcol)

    lane = jax.lax.broadcasted_iota(jnp.int32, (1, 1, 128), 2)
    out_ref[...] = jnp.where(lane == 0, nll_sum,
                             jnp.where(lane == 1, w_sum, 0.0)).astype(F32)


def kernel(inputs_ids, labels, labels_mask, valid_mask, params):
    B, S = inputs_ids.shape
    D = params['Wq'].shape[0]
    V, NL = params['chW'].shape
    FF = params['W1'].shape[1]
    H = 8
    DH = D // H
    CH = 64
    T = 512
    NS = S // T
    A1 = params['pos1'].shape[0]
    A2 = params['pos2'].shape[1]

    pos = jnp.concatenate([
        jnp.broadcast_to(params['pos1'], (A1, A2, D // 2)),
        jnp.broadcast_to(params['pos2'], (A1, A2, D // 2)),
    ], axis=-1).reshape(S, D)

    ids = inputs_ids.astype(jnp.int32)
    ids_m = ids.reshape(B * NS, T, 1)
    ids_h = jnp.pad(ids, ((0, 0), (CH, 0)))[:, :S].reshape(B, NS, T)[:, :, :CH]
    ids_h = ids_h.reshape(B * NS, CH, 1)
    pos_m = pos.reshape(NS, T, D)
    pos_h = jnp.pad(pos, ((CH, 0), (0, 0)))[:S].reshape(NS, T, D)[:, :CH]

    lbl_oh = (labels[..., None] == jnp.arange(NL)).astype(F32)
    lbl_oh = lbl_oh.reshape(B * NS, T, NL)
    w = ((valid_mask == 1) & (labels != 0)).astype(F32).reshape(B * NS, T, 1)

    r2 = lambda a: a.reshape(1, -1)
    p = params

    grid = (NS, B)
    in_specs = [
        pl.BlockSpec((1, T, 1), lambda s, b: (b * NS + s, 0, 0)),
        pl.BlockSpec((1, CH, 1), lambda s, b: (b * NS + s, 0, 0)),
        pl.BlockSpec((1, T, D), lambda s, b: (s, 0, 0)),
        pl.BlockSpec((1, CH, D), lambda s, b: (s, 0, 0)),
        pl.BlockSpec((V, D), lambda s, b: (0, 0)),
    ]
    const2d = lambda shape: pl.BlockSpec(shape, lambda s, b: (0, 0))
    bf = lambda a: a.astype(BF16)
    weights = [
        (bf(p['Wq']), (D, D)), (r2(p['bq']), (1, D)),
        (bf(p['Wk']), (D, D)), (r2(p['bk']), (1, D)),
        (bf(p['Wv']), (D, D)), (r2(p['bv']), (1, D)),
        (bf(p['Wo']), (D, D)), (r2(p['bo']), (1, D)),
        (r2(p['g1']), (1, D)), (r2(p['b1']), (1, D)),
        (r2(p['g2']), (1, D)), (r2(p['b2']), (1, D)),
        (bf(p['W1']), (D, FF)), (r2(p['bf1']), (1, FF)),
        (bf(p['W2']), (FF, D)), (r2(p['bf2']), (1, D)),
        (bf(p['lmW']), (D, V)), (r2(p['lmb']), (1, V)),
        (bf(p['l1W']), (V, V)), (r2(p['l1b']), (1, V)),
        (bf(p['chW']), (V, NL)), (r2(p['chb']), (1, NL)),
    ]
    in_specs += [const2d(shape) for _, shape in weights]
    in_specs += [
        pl.BlockSpec((1, T, NL), lambda s, b: (b * NS + s, 0, 0)),
        pl.BlockSpec((1, T, 1), lambda s, b: (b * NS + s, 0, 0)),
    ]

    out_spec = pl.BlockSpec((1, 1, 128), lambda s, b: (b * NS + s, 0, 0))

    fn = functools.partial(_fwd_kernel, T=T, CH=CH, H=H, DH=DH, V=V, NL=NL)
    partials = pl.pallas_call(
        fn,
        grid=grid,
        in_specs=in_specs,
        out_specs=out_spec,
        out_shape=jax.ShapeDtypeStruct((B * NS, 1, 128), F32),
    )(ids_m, ids_h, pos_m, pos_h, bf(p['emb']),
      *[a for a, _ in weights], lbl_oh, w)

    nll_tot = jnp.sum(partials[:, 0, 0])
    w_tot = jnp.sum(partials[:, 0, 1])
    return nll_tot / jnp.maximum(w_tot, 1.0)


# speedup vs baseline: 1.0945x; 1.0945x over previous
import functools

import jax
import jax.numpy as jnp
from jax.experimental import pallas as pl

F32 = jnp.float32
BF16 = jnp.bfloat16


def _dot(a, b):
    return jnp.dot(a.astype(BF16), b.astype(BF16), preferred_element_type=F32)


def _ln(x, g, b):
    m = x.mean(-1, keepdims=True)
    v = ((x - m) ** 2).mean(-1, keepdims=True)
    return (x - m) / jnp.sqrt(v + 1e-6) * g + b


def _fwd_kernel(
    ids_m_ref, ids_h_ref, pos_m_ref, pos_h_ref, emb_ref,
    wq_ref, bq_ref, wk_ref, bk_ref, wv_ref, bv_ref, wo_ref, bo_ref,
    g1_ref, b1_ref, g2_ref, b2_ref,
    w1_ref, bf1_ref, w2_ref, bf2_ref,
    lmw_ref, lmb_ref, l1w_ref, l1b_ref, chw_ref, chb_ref,
    lbl_oh_ref, w_ref,
    out_ref,
    *, T, CH, H, DH, V, NL,
):
    s_blk = pl.program_id(0)
    TE = T + CH

    ids_ext = jnp.concatenate([ids_h_ref[0], ids_m_ref[0]], axis=0)
    pos_ext = jnp.concatenate([pos_h_ref[0], pos_m_ref[0]], axis=0)
    onehot = (ids_ext == jax.lax.broadcasted_iota(jnp.int32, (TE, V), 1)
              ).astype(BF16)
    x_ext = _dot(onehot, emb_ref[...]) + pos_ext

    h1 = _ln(x_ext, g1_ref[...], b1_ref[...])
    q_all = _dot(h1[CH:], wq_ref[...]) + bq_ref[...]
    k_all = _dot(h1, wk_ref[...]) + bk_ref[...]
    v_all = _dot(h1, wv_ref[...]) + bv_ref[...]

    NCB = T // CH
    n_i = jax.lax.broadcasted_iota(jnp.int32, (NCB, CH, 2 * CH), 0)
    qi = jax.lax.broadcasted_iota(jnp.int32, (NCB, CH, 2 * CH), 1)
    ki = jax.lax.broadcasted_iota(jnp.int32, (NCB, CH, 2 * CH), 2)
    gchunk = s_blk * NCB + n_i
    mask = (ki <= qi + CH) & ((gchunk > 0) | (ki >= CH))

    scale = 1.0 / (DH ** 0.5)
    outs = []
    for h in range(H):
        sl = slice(h * DH, (h + 1) * DH)
        qh = q_all[:, sl].reshape(NCB, CH, DH).astype(BF16)
        kh = k_all[:, sl]
        vh = v_all[:, sl]
        kk = jnp.concatenate([kh[:T].reshape(NCB, CH, DH),
                              kh[CH:].reshape(NCB, CH, DH)], axis=1).astype(BF16)
        vv = jnp.concatenate([vh[:T].reshape(NCB, CH, DH),
                              vh[CH:].reshape(NCB, CH, DH)], axis=1).astype(BF16)
        sc = jax.lax.dot_general(qh, kk, (((2,), (2,)), ((0,), (0,))),
                                 preferred_element_type=F32) * scale
        sc = jnp.where(mask, sc, -1e9)
        mx = jnp.max(sc, axis=-1, keepdims=True)
        p = jnp.exp(sc - mx)
        z = jnp.sum(p, axis=-1, keepdims=True)
        oh_ = jax.lax.dot_general(p.astype(BF16), vv,
                                  (((2,), (1,)), ((0,), (0,))),
                                  preferred_element_type=F32)
        outs.append((oh_ / z).reshape(T, DH))
    o = jnp.concatenate(outs, axis=-1)

    x = x_ext[CH:] + _dot(o, wo_ref[...]) + bo_ref[...]

    h2 = _ln(x, g2_ref[...], b2_ref[...])
    ff = jnp.maximum(_dot(h2, w1_ref[...]) + bf1_ref[...], 0.0)
    x = x + _dot(ff, w2_ref[...]) + bf2_ref[...]

    lm = _dot(x, lmw_ref[...]) + lmb_ref[...]
    o1 = jnp.maximum(_dot(lm, l1w_ref[...]) + l1b_ref[...], 0.0)
    logits = _dot(o1, chw_ref[...]) + chb_ref[...]

    mx = jnp.max(logits, axis=-1, keepdims=True)
    lse = mx + jnp.log(jnp.sum(jnp.exp(logits - mx), axis=-1, keepdims=True))
    sel = jnp.sum(lbl_oh_ref[0] * logits, axis=-1, keepdims=True)
    wcol = w_ref[0]
    nll_sum = jnp.sum((lse - sel) * wcol)
    w_sum = jnp.sum(wcol)

    lane = jax.lax.broadcasted_iota(jnp.int32, (1, 1, 128), 2)
    out_ref[...] = jnp.where(lane == 0, nll_sum,
                             jnp.where(lane == 1, w_sum, 0.0)).astype(F32)


def kernel(inputs_ids, labels, labels_mask, valid_mask, params):
    B, S = inputs_ids.shape
    D = params['Wq'].shape[0]
    V, NL = params['chW'].shape
    FF = params['W1'].shape[1]
    H = 8
    DH = D // H
    CH = 64
    T = 512
    NS = S // T
    A1 = params['pos1'].shape[0]
    A2 = params['pos2'].shape[1]

    pos = jnp.concatenate([
        jnp.broadcast_to(params['pos1'], (A1, A2, D // 2)),
        jnp.broadcast_to(params['pos2'], (A1, A2, D // 2)),
    ], axis=-1).reshape(S, D)

    ids = inputs_ids.astype(jnp.int32)
    ids_m = ids.reshape(B * NS, T, 1)
    ids_h = jnp.pad(ids, ((0, 0), (CH, 0)))[:, :S].reshape(B, NS, T)[:, :, :CH]
    ids_h = ids_h.reshape(B * NS, CH, 1)
    pos_m = pos.reshape(NS, T, D)
    pos_h = jnp.pad(pos, ((CH, 0), (0, 0)))[:S].reshape(NS, T, D)[:, :CH]

    lbl_oh = (labels[..., None] == jnp.arange(NL)).astype(F32)
    lbl_oh = lbl_oh.reshape(B * NS, T, NL)
    w = ((valid_mask == 1) & (labels != 0)).astype(F32).reshape(B * NS, T, 1)

    r2 = lambda a: a.reshape(1, -1)
    p = params

    grid = (NS, B)
    in_specs = [
        pl.BlockSpec((1, T, 1), lambda s, b: (b * NS + s, 0, 0)),
        pl.BlockSpec((1, CH, 1), lambda s, b: (b * NS + s, 0, 0)),
        pl.BlockSpec((1, T, D), lambda s, b: (s, 0, 0)),
        pl.BlockSpec((1, CH, D), lambda s, b: (s, 0, 0)),
        pl.BlockSpec((V, D), lambda s, b: (0, 0)),
    ]
    const2d = lambda shape: pl.BlockSpec(shape, lambda s, b: (0, 0))
    bf = lambda a: a.astype(BF16)
    weights = [
        (bf(p['Wq']), (D, D)), (r2(p['bq']), (1, D)),
        (bf(p['Wk']), (D, D)), (r2(p['bk']), (1, D)),
        (bf(p['Wv']), (D, D)), (r2(p['bv']), (1, D)),
        (bf(p['Wo']), (D, D)), (r2(p['bo']), (1, D)),
        (r2(p['g1']), (1, D)), (r2(p['b1']), (1, D)),
        (r2(p['g2']), (1, D)), (r2(p['b2']), (1, D)),
        (bf(p['W1']), (D, FF)), (r2(p['bf1']), (1, FF)),
        (bf(p['W2']), (FF, D)), (r2(p['bf2']), (1, D)),
        (bf(p['lmW']), (D, V)), (r2(p['lmb']), (1, V)),
        (bf(p['l1W']), (V, V)), (r2(p['l1b']), (1, V)),
        (bf(p['chW']), (V, NL)), (r2(p['chb']), (1, NL)),
    ]
    in_specs += [const2d(shape) for _, shape in weights]
    in_specs += [
        pl.BlockSpec((1, T, NL), lambda s, b: (b * NS + s, 0, 0)),
        pl.BlockSpec((1, T, 1), lambda s, b: (b * NS + s, 0, 0)),
    ]

    out_spec = pl.BlockSpec((1, 1, 128), lambda s, b: (b * NS + s, 0, 0))

    fn = functools.partial(_fwd_kernel, T=T, CH=CH, H=H, DH=DH, V=V, NL=NL)
    partials = pl.pallas_call(
        fn,
        grid=grid,
        in_specs=in_specs,
        out_specs=out_spec,
        out_shape=jax.ShapeDtypeStruct((B * NS, 1, 128), F32),
    )(ids_m, ids_h, pos_m, pos_h, bf(p['emb']),
      *[a for a, _ in weights], lbl_oh, w)

    nll_tot = jnp.sum(partials[:, 0, 0])
    w_tot = jnp.sum(partials[:, 0, 1])
    return nll_tot / jnp.maximum(w_tot, 1.0)


# speedup vs baseline: 1.2261x; 1.1202x over previous
import functools

import jax
import jax.numpy as jnp
from jax.experimental import pallas as pl

F32 = jnp.float32
BF16 = jnp.bfloat16


def _dot(a, b):
    return jnp.dot(a.astype(BF16), b.astype(BF16), preferred_element_type=F32)


def _ln(x, g, b):
    m = x.mean(-1, keepdims=True)
    v = ((x - m) ** 2).mean(-1, keepdims=True)
    return (x - m) / jnp.sqrt(v + 1e-6) * g + b


def _fwd_kernel(
    ids_m_ref, ids_h_ref, pos1c_ref, pos1p_ref, pos2_ref, emb_ref,
    wq_ref, bq_ref, wk_ref, bk_ref, wv_ref, bv_ref, wo_ref, bo_ref,
    g1_ref, b1_ref, g2_ref, b2_ref,
    w1_ref, bf1_ref, w2_ref, bf2_ref,
    lmw_ref, lmb_ref, l1w_ref, l1b_ref, chw_ref, chb_ref,
    lbl_oh_ref, w_ref,
    out_ref,
    *, T, CH, H, DH, V, NL,
):
    s_blk = pl.program_id(0)
    TE = T + CH

    Dh2 = pos2_ref.shape[1]
    p2 = pos2_ref[...]
    pos_main = jnp.concatenate(
        [jnp.broadcast_to(pos1c_ref[0], (T, Dh2)), p2], axis=-1)
    pos_halo = jnp.concatenate(
        [jnp.broadcast_to(pos1p_ref[0], (CH, Dh2)), p2[T - CH:]], axis=-1)
    ids_ext = jnp.concatenate([ids_h_ref[0], ids_m_ref[0]], axis=0)
    pos_ext = jnp.concatenate([pos_halo, pos_main], axis=0)
    onehot = (ids_ext == jax.lax.broadcasted_iota(jnp.int32, (TE, V), 1)
              ).astype(BF16)
    x_ext = _dot(onehot, emb_ref[...]) + pos_ext

    h1 = _ln(x_ext, g1_ref[...], b1_ref[...])
    q_all = _dot(h1[CH:], wq_ref[...]) + bq_ref[...]
    k_all = _dot(h1, wk_ref[...]) + bk_ref[...]
    v_all = _dot(h1, wv_ref[...]) + bv_ref[...]

    NCB = T // CH
    n_i = jax.lax.broadcasted_iota(jnp.int32, (H * NCB, CH, 2 * CH), 0) % NCB
    qi = jax.lax.broadcasted_iota(jnp.int32, (H * NCB, CH, 2 * CH), 1)
    ki = jax.lax.broadcasted_iota(jnp.int32, (H * NCB, CH, 2 * CH), 2)
    gchunk = s_blk * NCB + n_i
    mask = (ki <= qi + CH) & ((gchunk > 0) | (ki >= CH))

    scale = 1.0 / (DH ** 0.5)
    qb = q_all.astype(BF16)
    kb = k_all.astype(BF16)
    vb = v_all.astype(BF16)
    q4 = jnp.concatenate(
        [qb[:, h * DH:(h + 1) * DH].reshape(NCB, CH, DH) for h in range(H)], 0)
    kk4 = jnp.concatenate(
        [jnp.concatenate([kb[:T, h * DH:(h + 1) * DH].reshape(NCB, CH, DH),
                          kb[CH:, h * DH:(h + 1) * DH].reshape(NCB, CH, DH)],
                         axis=1) for h in range(H)], 0)
    vv4 = jnp.concatenate(
        [jnp.concatenate([vb[:T, h * DH:(h + 1) * DH].reshape(NCB, CH, DH),
                          vb[CH:, h * DH:(h + 1) * DH].reshape(NCB, CH, DH)],
                         axis=1) for h in range(H)], 0)
    sc = jax.lax.dot_general(q4, kk4, (((2,), (2,)), ((0,), (0,))),
                             preferred_element_type=F32) * scale
    sc = jnp.where(mask, sc, -1e9)
    mx = jnp.max(sc, axis=-1, keepdims=True)
    pr = jnp.exp(sc - mx)
    z = jnp.sum(pr, axis=-1, keepdims=True)
    o4 = jax.lax.dot_general(pr.astype(BF16), vv4,
                             (((2,), (1,)), ((0,), (0,))),
                             preferred_element_type=F32) / z
    o = jnp.concatenate(
        [o4[h * NCB:(h + 1) * NCB].reshape(T, DH) for h in range(H)], axis=-1)

    x = x_ext[CH:] + _dot(o, wo_ref[...]) + bo_ref[...]

    h2 = _ln(x, g2_ref[...], b2_ref[...])
    ff = jnp.maximum(_dot(h2, w1_ref[...]) + bf1_ref[...], 0.0)
    x = x + _dot(ff, w2_ref[...]) + bf2_ref[...]

    lm = _dot(x, lmw_ref[...]) + lmb_ref[...]
    o1 = jnp.maximum(_dot(lm, l1w_ref[...]) + l1b_ref[...], 0.0)
    logits = _dot(o1, chw_ref[...]) + chb_ref[...]

    mx = jnp.max(logits, axis=-1, keepdims=True)
    lse = mx + jnp.log(jnp.sum(jnp.exp(logits - mx), axis=-1, keepdims=True))
    sel = jnp.sum(lbl_oh_ref[0] * logits, axis=-1, keepdims=True)
    wcol = w_ref[0]
    nll_sum = jnp.sum((lse - sel) * wcol)
    w_sum = jnp.sum(wcol)

    lane = jax.lax.broadcasted_iota(jnp.int32, (1, 1, 128), 2)
    out_ref[...] = jnp.where(lane == 0, nll_sum,
                             jnp.where(lane == 1, w_sum, 0.0)).astype(F32)


def kernel(inputs_ids, labels, labels_mask, valid_mask, params):
    B, S = inputs_ids.shape
    D = params['Wq'].shape[0]
    V, NL = params['chW'].shape
    FF = params['W1'].shape[1]
    H = 8
    DH = D // H
    CH = 64
    T = 512
    NS = S // T
    A1 = params['pos1'].shape[0]
    A2 = params['pos2'].shape[1]

    assert T == A2 and A1 * A2 == S

    ids = inputs_ids.astype(jnp.int32)
    ids_m = ids.reshape(B * NS, T, 1)
    ids_h = jnp.pad(ids, ((0, 0), (CH, 0)))[:, :S].reshape(B, NS, T)[:, :, :CH]
    ids_h = ids_h.reshape(B * NS, CH, 1)
    pos2r = params['pos2'].reshape(A2, D // 2)

    lbl_oh = (labels[..., None] == jnp.arange(NL)).astype(F32)
    lbl_oh = lbl_oh.reshape(B * NS, T, NL)
    w = ((valid_mask == 1) & (labels != 0)).astype(F32).reshape(B * NS, T, 1)

    r2 = lambda a: a.reshape(1, -1)
    p = params

    grid = (NS, B)
    in_specs = [
        pl.BlockSpec((1, T, 1), lambda s, b: (b * NS + s, 0, 0)),
        pl.BlockSpec((1, CH, 1), lambda s, b: (b * NS + s, 0, 0)),
        pl.BlockSpec((1, 1, D // 2), lambda s, b: (s, 0, 0)),
        pl.BlockSpec((1, 1, D // 2), lambda s, b: (jnp.maximum(s - 1, 0), 0, 0)),
        pl.BlockSpec((A2, D // 2), lambda s, b: (0, 0)),
        pl.BlockSpec((V, D), lambda s, b: (0, 0)),
    ]
    const2d = lambda shape: pl.BlockSpec(shape, lambda s, b: (0, 0))
    bf = lambda a: a.astype(BF16)
    weights = [
        (bf(p['Wq']), (D, D)), (r2(p['bq']), (1, D)),
        (bf(p['Wk']), (D, D)), (r2(p['bk']), (1, D)),
        (bf(p['Wv']), (D, D)), (r2(p['bv']), (1, D)),
        (bf(p['Wo']), (D, D)), (r2(p['bo']), (1, D)),
        (r2(p['g1']), (1, D)), (r2(p['b1']), (1, D)),
        (r2(p['g2']), (1, D)), (r2(p['b2']), (1, D)),
        (bf(p['W1']), (D, FF)), (r2(p['bf1']), (1, FF)),
        (bf(p['W2']), (FF, D)), (r2(p['bf2']), (1, D)),
        (bf(p['lmW']), (D, V)), (r2(p['lmb']), (1, V)),
        (bf(p['l1W']), (V, V)), (r2(p['l1b']), (1, V)),
        (bf(p['chW']), (V, NL)), (r2(p['chb']), (1, NL)),
    ]
    in_specs += [const2d(shape) for _, shape in weights]
    in_specs += [
        pl.BlockSpec((1, T, NL), lambda s, b: (b * NS + s, 0, 0)),
        pl.BlockSpec((1, T, 1), lambda s, b: (b * NS + s, 0, 0)),
    ]

    out_spec = pl.BlockSpec((1, 1, 128), lambda s, b: (b * NS + s, 0, 0))

    fn = functools.partial(_fwd_kernel, T=T, CH=CH, H=H, DH=DH, V=V, NL=NL)
    partials = pl.pallas_call(
        fn,
        grid=grid,
        in_specs=in_specs,
        out_specs=out_spec,
        out_shape=jax.ShapeDtypeStruct((B * NS, 1, 128), F32),
    )(ids_m, ids_h, p['pos1'], p['pos1'], pos2r, bf(p['emb']),
      *[a for a, _ in weights], lbl_oh, w)

    nll_tot = jnp.sum(partials[:, 0, 0])
    w_tot = jnp.sum(partials[:, 0, 1])
    return nll_tot / jnp.maximum(w_tot, 1.0)


# speedup vs baseline: 1.2283x; 1.0018x over previous
import functools

import jax
import jax.numpy as jnp
from jax.experimental import pallas as pl
from jax.experimental.pallas import tpu as pltpu

F32 = jnp.float32
BF16 = jnp.bfloat16


def _dot(a, b):
    return jnp.dot(a.astype(BF16), b.astype(BF16), preferred_element_type=F32)


def _ln(x, g, b):
    m = x.mean(-1, keepdims=True)
    v = ((x - m) ** 2).mean(-1, keepdims=True)
    return (x - m) / jnp.sqrt(v + 1e-6) * g + b


def _fwd_kernel(
    ids_m_ref, ids_h_ref, pos1c_ref, pos1p_ref, pos2_ref, emb_ref,
    wq_ref, bq_ref, wk_ref, bk_ref, wv_ref, bv_ref, wo_ref, bo_ref,
    g1_ref, b1_ref, g2_ref, b2_ref,
    w1_ref, bf1_ref, w2_ref, bf2_ref,
    lmw_ref, lmb_ref, l1w_ref, l1b_ref, chw_ref, chb_ref,
    lbl_oh_ref, w_ref,
    out_ref,
    *, T, CH, H, DH, V, NL,
):
    s_blk = pl.program_id(0)
    TE = T + CH

    Dh2 = pos2_ref.shape[1]
    p2 = pos2_ref[...]
    pos_main = jnp.concatenate(
        [jnp.broadcast_to(pos1c_ref[0], (T, Dh2)), p2], axis=-1)
    pos_halo = jnp.concatenate(
        [jnp.broadcast_to(pos1p_ref[0], (CH, Dh2)), p2[T - CH:]], axis=-1)
    ids_ext = jnp.concatenate([ids_h_ref[0], ids_m_ref[0]], axis=0)
    pos_ext = jnp.concatenate([pos_halo, pos_main], axis=0)
    onehot = (ids_ext == jax.lax.broadcasted_iota(jnp.int32, (TE, V), 1)
              ).astype(BF16)
    x_ext = _dot(onehot, emb_ref[...]) + pos_ext

    h1 = _ln(x_ext, g1_ref[...], b1_ref[...])
    q_all = _dot(h1[CH:], wq_ref[...]) + bq_ref[...]
    k_all = _dot(h1, wk_ref[...]) + bk_ref[...]
    v_all = _dot(h1, wv_ref[...]) + bv_ref[...]

    NCB = T // CH
    n_i = jax.lax.broadcasted_iota(jnp.int32, (H * NCB, CH, 2 * CH), 0) % NCB
    qi = jax.lax.broadcasted_iota(jnp.int32, (H * NCB, CH, 2 * CH), 1)
    ki = jax.lax.broadcasted_iota(jnp.int32, (H * NCB, CH, 2 * CH), 2)
    gchunk = s_blk * NCB + n_i
    mask = (ki <= qi + CH) & ((gchunk > 0) | (ki >= CH))

    scale = 1.0 / (DH ** 0.5)
    qb = (q_all * scale).astype(BF16)
    kb = k_all.astype(BF16)
    vb = v_all.astype(BF16)
    q4 = jnp.concatenate(
        [qb[:, h * DH:(h + 1) * DH].reshape(NCB, CH, DH) for h in range(H)], 0)
    kk4 = jnp.concatenate(
        [jnp.concatenate([kb[:T, h * DH:(h + 1) * DH].reshape(NCB, CH, DH),
                          kb[CH:, h * DH:(h + 1) * DH].reshape(NCB, CH, DH)],
                         axis=1) for h in range(H)], 0)
    vv4 = jnp.concatenate(
        [jnp.concatenate([vb[:T, h * DH:(h + 1) * DH].reshape(NCB, CH, DH),
                          vb[CH:, h * DH:(h + 1) * DH].reshape(NCB, CH, DH)],
                         axis=1) for h in range(H)], 0)
    sc = jax.lax.dot_general(q4, kk4, (((2,), (2,)), ((0,), (0,))),
                             preferred_element_type=F32)
    sc = jnp.where(mask, sc, -1e9)
    mx = jnp.max(sc, axis=-1, keepdims=True)
    pr = jnp.exp(sc - mx)
    z = jnp.sum(pr, axis=-1, keepdims=True)
    o4 = jax.lax.dot_general(pr.astype(BF16), vv4,
                             (((2,), (1,)), ((0,), (0,))),
                             preferred_element_type=F32) / z
    o = jnp.concatenate(
        [o4[h * NCB:(h + 1) * NCB].reshape(T, DH) for h in range(H)], axis=-1)

    x = x_ext[CH:] + _dot(o, wo_ref[...]) + bo_ref[...]

    h2 = _ln(x, g2_ref[...], b2_ref[...])
    ff = jnp.maximum(_dot(h2, w1_ref[...]) + bf1_ref[...], 0.0)
    x = x + _dot(ff, w2_ref[...]) + bf2_ref[...]

    lm = _dot(x, lmw_ref[...]) + lmb_ref[...]
    o1 = jnp.maximum(_dot(lm, l1w_ref[...]) + l1b_ref[...], 0.0)
    logits = _dot(o1, chw_ref[...]) + chb_ref[...]

    mx = jnp.max(logits, axis=-1, keepdims=True)
    lse = mx + jnp.log(jnp.sum(jnp.exp(logits - mx), axis=-1, keepdims=True))
    sel = jnp.sum(lbl_oh_ref[0] * logits, axis=-1, keepdims=True)
    wcol = w_ref[0]
    nll_sum = jnp.sum((lse - sel) * wcol)
    w_sum = jnp.sum(wcol)

    lane = jax.lax.broadcasted_iota(jnp.int32, (1, 1, 128), 2)
    out_ref[...] = jnp.where(lane == 0, nll_sum,
                             jnp.where(lane == 1, w_sum, 0.0)).astype(F32)


def kernel(inputs_ids, labels, labels_mask, valid_mask, params):
    B, S = inputs_ids.shape
    D = params['Wq'].shape[0]
    V, NL = params['chW'].shape
    FF = params['W1'].shape[1]
    H = 8
    DH = D // H
    CH = 64
    T = 512
    NS = S // T
    A1 = params['pos1'].shape[0]
    A2 = params['pos2'].shape[1]

    assert T == A2 and A1 * A2 == S

    ids = inputs_ids.astype(jnp.int32)
    ids_m = ids.reshape(B * NS, T, 1)
    ids_h = jnp.pad(ids, ((0, 0), (CH, 0)))[:, :S].reshape(B, NS, T)[:, :, :CH]
    ids_h = ids_h.reshape(B * NS, CH, 1)
    pos2r = params['pos2'].reshape(A2, D // 2)

    lbl_oh = (labels[..., None] == jnp.arange(NL)).astype(F32)
    lbl_oh = lbl_oh.reshape(B * NS, T, NL)
    w = ((valid_mask == 1) & (labels != 0)).astype(F32).reshape(B * NS, T, 1)

    r2 = lambda a: a.reshape(1, -1)
    p = params

    grid = (NS, B)
    in_specs = [
        pl.BlockSpec((1, T, 1), lambda s, b: (b * NS + s, 0, 0)),
        pl.BlockSpec((1, CH, 1), lambda s, b: (b * NS + s, 0, 0)),
        pl.BlockSpec((1, 1, D // 2), lambda s, b: (s, 0, 0)),
        pl.BlockSpec((1, 1, D // 2), lambda s, b: (jnp.maximum(s - 1, 0), 0, 0)),
        pl.BlockSpec((A2, D // 2), lambda s, b: (0, 0)),
        pl.BlockSpec((V, D), lambda s, b: (0, 0)),
    ]
    const2d = lambda shape: pl.BlockSpec(shape, lambda s, b: (0, 0))
    bf = lambda a: a.astype(BF16)
    weights = [
        (bf(p['Wq']), (D, D)), (r2(p['bq']), (1, D)),
        (bf(p['Wk']), (D, D)), (r2(p['bk']), (1, D)),
        (bf(p['Wv']), (D, D)), (r2(p['bv']), (1, D)),
        (bf(p['Wo']), (D, D)), (r2(p['bo']), (1, D)),
        (r2(p['g1']), (1, D)), (r2(p['b1']), (1, D)),
        (r2(p['g2']), (1, D)), (r2(p['b2']), (1, D)),
        (bf(p['W1']), (D, FF)), (r2(p['bf1']), (1, FF)),
        (bf(p['W2']), (FF, D)), (r2(p['bf2']), (1, D)),
        (bf(p['lmW']), (D, V)), (r2(p['lmb']), (1, V)),
        (bf(p['l1W']), (V, V)), (r2(p['l1b']), (1, V)),
        (bf(p['chW']), (V, NL)), (r2(p['chb']), (1, NL)),
    ]
    in_specs += [const2d(shape) for _, shape in weights]
    in_specs += [
        pl.BlockSpec((1, T, NL), lambda s, b: (b * NS + s, 0, 0)),
        pl.BlockSpec((1, T, 1), lambda s, b: (b * NS + s, 0, 0)),
    ]

    out_spec = pl.BlockSpec((1, 1, 128), lambda s, b: (b * NS + s, 0, 0))

    fn = functools.partial(_fwd_kernel, T=T, CH=CH, H=H, DH=DH, V=V, NL=NL)
    partials = pl.pallas_call(
        fn,
        grid=grid,
        in_specs=in_specs,
        out_specs=out_spec,
        out_shape=jax.ShapeDtypeStruct((B * NS, 1, 128), F32),
        compiler_params=pltpu.CompilerParams(
            dimension_semantics=("parallel", "parallel")),
    )(ids_m, ids_h, p['pos1'], p['pos1'], pos2r, bf(p['emb']),
      *[a for a, _ in weights], lbl_oh, w)

    nll_tot = jnp.sum(partials[:, 0, 0])
    w_tot = jnp.sum(partials[:, 0, 1])
    return nll_tot / jnp.maximum(w_tot, 1.0)


# speedup vs baseline: 1.3233x; 1.0773x over previous
import functools

import jax
import jax.numpy as jnp
from jax.experimental import pallas as pl
from jax.experimental.pallas import tpu as pltpu

F32 = jnp.float32
BF16 = jnp.bfloat16


def _dot(a, b):
    return jnp.dot(a.astype(BF16), b.astype(BF16), preferred_element_type=F32)


def _ln(x, g, b):
    m = x.mean(-1, keepdims=True)
    v = ((x - m) ** 2).mean(-1, keepdims=True)
    return (x - m) / jnp.sqrt(v + 1e-6) * g + b


def _stream(ids_m_ref, ids_h_ref, lbl_oh_ref, w_ref, pos_ext, mask, refs,
            *, T, CH, H, DH, V, NL):
    (emb_ref, wq_ref, bq_ref, wk_ref, bk_ref, wv_ref, bv_ref, wo_ref, bo_ref,
     g1_ref, b1_ref, g2_ref, b2_ref, w1_ref, bf1_ref, w2_ref, bf2_ref,
     lmw_ref, lmb_ref, l1w_ref, l1b_ref, chw_ref, chb_ref) = refs
    TE = T + CH
    NCB = T // CH

    ids_ext = jnp.concatenate([ids_h_ref[0], ids_m_ref[0]], axis=0)
    onehot = (ids_ext == jax.lax.broadcasted_iota(jnp.int32, (TE, V), 1)
              ).astype(BF16)
    x_ext = _dot(onehot, emb_ref[...]) + pos_ext

    h1 = _ln(x_ext, g1_ref[...], b1_ref[...])
    h1b = h1.astype(BF16)
    q_all = _dot(h1b[CH:], wq_ref[...]) + bq_ref[...]
    k_all = _dot(h1b, wk_ref[...]) + bk_ref[...]
    v_all = _dot(h1b, wv_ref[...]) + bv_ref[...]

    scale = 1.0 / (DH ** 0.5)
    qb = (q_all * scale).astype(BF16)
    kb = k_all.astype(BF16)
    vb = v_all.astype(BF16)
    q4 = jnp.concatenate(
        [qb[:, h * DH:(h + 1) * DH].reshape(NCB, CH, DH) for h in range(H)], 0)
    kk4 = jnp.concatenate(
        [jnp.concatenate([kb[:T, h * DH:(h + 1) * DH].reshape(NCB, CH, DH),
                          kb[CH:, h * DH:(h + 1) * DH].reshape(NCB, CH, DH)],
                         axis=1) for h in range(H)], 0)
    vv4 = jnp.concatenate(
        [jnp.concatenate([vb[:T, h * DH:(h + 1) * DH].reshape(NCB, CH, DH),
                          vb[CH:, h * DH:(h + 1) * DH].reshape(NCB, CH, DH)],
                         axis=1) for h in range(H)], 0)
    sc = jax.lax.dot_general(q4, kk4, (((2,), (2,)), ((0,), (0,))),
                             preferred_element_type=F32)
    sc = jnp.where(mask, sc, -1e9)
    mx = jnp.max(sc, axis=-1, keepdims=True)
    pr = jnp.exp(sc - mx)
    z = jnp.sum(pr, axis=-1, keepdims=True)
    o4 = jax.lax.dot_general(pr.astype(BF16), vv4,
                             (((2,), (1,)), ((0,), (0,))),
                             preferred_element_type=F32) / z
    o = jnp.concatenate(
        [o4[h * NCB:(h + 1) * NCB].reshape(T, DH) for h in range(H)], axis=-1)

    x = x_ext[CH:] + _dot(o, wo_ref[...]) + bo_ref[...]

    h2 = _ln(x, g2_ref[...], b2_ref[...])
    ff = jnp.maximum(_dot(h2, w1_ref[...]) + bf1_ref[...], 0.0)
    x = x + _dot(ff, w2_ref[...]) + bf2_ref[...]

    lm = _dot(x, lmw_ref[...]) + lmb_ref[...]
    o1 = jnp.maximum(_dot(lm, l1w_ref[...]) + l1b_ref[...], 0.0)
    logits = _dot(o1, chw_ref[...]) + chb_ref[...]

    mx2 = jnp.max(logits, axis=-1, keepdims=True)
    lse = mx2 + jnp.log(jnp.sum(jnp.exp(logits - mx2), axis=-1, keepdims=True))
    sel = jnp.sum(lbl_oh_ref[0] * logits, axis=-1, keepdims=True)
    wcol = w_ref[0]
    return jnp.sum((lse - sel) * wcol), jnp.sum(wcol)


def _fwd_kernel(
    ids_m1_ref, ids_h1_ref, lbl_oh1_ref, w1m_ref,
    ids_m2_ref, ids_h2_ref, lbl_oh2_ref, w2m_ref,
    pos1c_ref, pos1p_ref, pos2_ref, emb_ref,
    wq_ref, bq_ref, wk_ref, bk_ref, wv_ref, bv_ref, wo_ref, bo_ref,
    g1_ref, b1_ref, g2_ref, b2_ref,
    w1_ref, bf1_ref, w2_ref, bf2_ref,
    lmw_ref, lmb_ref, l1w_ref, l1b_ref, chw_ref, chb_ref,
    out1_ref, out2_ref,
    *, T, CH, H, DH, V, NL,
):
    s_blk = pl.program_id(0)
    NCB = T // CH

    Dh2 = pos2_ref.shape[1]
    p2 = pos2_ref[...]
    pos_main = jnp.concatenate(
        [jnp.broadcast_to(pos1c_ref[0], (T, Dh2)), p2], axis=-1)
    pos_halo = jnp.concatenate(
        [jnp.broadcast_to(pos1p_ref[0], (CH, Dh2)), p2[T - CH:]], axis=-1)
    pos_ext = jnp.concatenate([pos_halo, pos_main], axis=0)

    n_i = jax.lax.broadcasted_iota(jnp.int32, (H * NCB, CH, 2 * CH), 0) % NCB
    qi = jax.lax.broadcasted_iota(jnp.int32, (H * NCB, CH, 2 * CH), 1)
    ki = jax.lax.broadcasted_iota(jnp.int32, (H * NCB, CH, 2 * CH), 2)
    gchunk = s_blk * NCB + n_i
    mask = (ki <= qi + CH) & ((gchunk > 0) | (ki >= CH))

    refs = (emb_ref, wq_ref, bq_ref, wk_ref, bk_ref, wv_ref, bv_ref, wo_ref,
            bo_ref, g1_ref, b1_ref, g2_ref, b2_ref, w1_ref, bf1_ref, w2_ref,
            bf2_ref, lmw_ref, lmb_ref, l1w_ref, l1b_ref, chw_ref, chb_ref)
    kw = dict(T=T, CH=CH, H=H, DH=DH, V=V, NL=NL)
    n1, w1s = _stream(ids_m1_ref, ids_h1_ref, lbl_oh1_ref, w1m_ref,
                      pos_ext, mask, refs, **kw)
    n2, w2s = _stream(ids_m2_ref, ids_h2_ref, lbl_oh2_ref, w2m_ref,
                      pos_ext, mask, refs, **kw)

    lane = jax.lax.broadcasted_iota(jnp.int32, (1, 1, 128), 2)
    out1_ref[...] = jnp.where(lane == 0, n1,
                              jnp.where(lane == 1, w1s, 0.0)).astype(F32)
    out2_ref[...] = jnp.where(lane == 0, n2,
                              jnp.where(lane == 1, w2s, 0.0)).astype(F32)


def kernel(inputs_ids, labels, labels_mask, valid_mask, params):
    B, S = inputs_ids.shape
    D = params['Wq'].shape[0]
    V, NL = params['chW'].shape
    FF = params['W1'].shape[1]
    H = 8
    DH = D // H
    CH = 64
    A1 = params['pos1'].shape[0]
    A2 = params['pos2'].shape[1]
    T = A2
    NS = S // T
    B2 = B // 2

    assert T == A2 and A1 * A2 == S and B % 2 == 0

    ids = inputs_ids.astype(jnp.int32)
    ids_m = ids.reshape(B * NS, T, 1)
    ids_h = jnp.pad(ids, ((0, 0), (CH, 0)))[:, :S].reshape(B, NS, T)[:, :, :CH]
    ids_h = ids_h.reshape(B * NS, CH, 1)
    pos2r = params['pos2'].reshape(A2, D // 2)

    lbl_oh = (labels[..., None] == jnp.arange(NL)).astype(F32)
    lbl_oh = lbl_oh.reshape(B * NS, T, NL)
    w = ((valid_mask == 1) & (labels != 0)).astype(F32).reshape(B * NS, T, 1)

    r2 = lambda a: a.reshape(1, -1)
    bf = lambda a: a.astype(BF16)
    p = params

    grid = (NS, B2)
    i1 = lambda s, b: (b * NS + s, 0, 0)
    i2 = lambda s, b: ((b + B2) * NS + s, 0, 0)
    stream_specs = lambda i: [
        pl.BlockSpec((1, T, 1), i),
        pl.BlockSpec((1, CH, 1), i),
        pl.BlockSpec((1, T, NL), i),
        pl.BlockSpec((1, T, 1), i),
    ]
    in_specs = stream_specs(i1) + stream_specs(i2) + [
        pl.BlockSpec((1, 1, D // 2), lambda s, b: (s, 0, 0)),
        pl.BlockSpec((1, 1, D // 2), lambda s, b: (jnp.maximum(s - 1, 0), 0, 0)),
        pl.BlockSpec((A2, D // 2), lambda s, b: (0, 0)),
        pl.BlockSpec((V, D), lambda s, b: (0, 0)),
    ]
    const2d = lambda shape: pl.BlockSpec(shape, lambda s, b: (0, 0))
    weights = [
        (bf(p['Wq']), (D, D)), (r2(p['bq']), (1, D)),
        (bf(p['Wk']), (D, D)), (r2(p['bk']), (1, D)),
        (bf(p['Wv']), (D, D)), (r2(p['bv']), (1, D)),
        (bf(p['Wo']), (D, D)), (r2(p['bo']), (1, D)),
        (r2(p['g1']), (1, D)), (r2(p['b1']), (1, D)),
        (r2(p['g2']), (1, D)), (r2(p['b2']), (1, D)),
        (bf(p['W1']), (D, FF)), (r2(p['bf1']), (1, FF)),
        (bf(p['W2']), (FF, D)), (r2(p['bf2']), (1, D)),
        (bf(p['lmW']), (D, V)), (r2(p['lmb']), (1, V)),
        (bf(p['l1W']), (V, V)), (r2(p['l1b']), (1, V)),
        (bf(p['chW']), (V, NL)), (r2(p['chb']), (1, NL)),
    ]
    in_specs += [const2d(shape) for _, shape in weights]

    out_spec = pl.BlockSpec((1, 1, 128), lambda s, b: (b * NS + s, 0, 0))

    fn = functools.partial(_fwd_kernel, T=T, CH=CH, H=H, DH=DH, V=V, NL=NL)
    p1, p2 = pl.pallas_call(
        fn,
        grid=grid,
        in_specs=in_specs,
        out_specs=[out_spec, out_spec],
        out_shape=[jax.ShapeDtypeStruct((B2 * NS, 1, 128), F32)] * 2,
        compiler_params=pltpu.CompilerParams(
            dimension_semantics=("parallel", "parallel")),
    )(ids_m, ids_h, lbl_oh, w,
      ids_m, ids_h, lbl_oh, w,
      p['pos1'], p['pos1'], pos2r, bf(p['emb']),
      *[a for a, _ in weights])

    nll_tot = jnp.sum(p1[:, 0, 0]) + jnp.sum(p2[:, 0, 0])
    w_tot = jnp.sum(p1[:, 0, 1]) + jnp.sum(p2[:, 0, 1])
    return nll_tot / jnp.maximum(w_tot, 1.0)
